# baseline (speedup 1.0000x reference)
import jax
import jax.numpy as jnp
from jax import lax
from jax.experimental import pallas as pl
from jax.experimental.pallas import tpu as pltpu
from jax.experimental.pallas import tpu_sc as plsc

BATCH = 4
SEQ_LEN = 8192
D_MODEL = 1024

NUM_CORES = 2
NUM_SUBCORES = 16
NUM_WORKERS = NUM_CORES * NUM_SUBCORES
ROWS_PER_WORKER = SEQ_LEN // NUM_WORKERS
CHUNK_ROWS = 32
NUM_CHUNKS = ROWS_PER_WORKER // CHUNK_ROWS
LANES = 16
GROUPS_PER_ROW = D_MODEL // LANES
GROUPS_PER_CHUNK = CHUNK_ROWS * GROUPS_PER_ROW


def _sc_body(x_hbm, pe_hbm, out_hbm, pe_buf, x_buf, sem):
    del sem
    wid = lax.axis_index("s") * NUM_CORES + lax.axis_index("c")
    base = wid * ROWS_PER_WORKER

    for j in range(NUM_CHUNKS):
        row = base + j * CHUNK_ROWS
        pltpu.sync_copy(pe_hbm.at[pl.ds(row, CHUNK_ROWS)], pe_buf)
        for b in range(BATCH):
            pltpu.sync_copy(x_hbm.at[b, pl.ds(row, CHUNK_ROWS)], x_buf)

            @plsc.parallel_loop(0, GROUPS_PER_CHUNK, unroll=8)
            def _(g):
                r = g >> 6
                k = (g & (GROUPS_PER_ROW - 1)) * LANES
                x_buf[r, pl.ds(k, LANES)] = (
                    x_buf[r, pl.ds(k, LANES)] + pe_buf[r, pl.ds(k, LANES)]
                )

            pltpu.sync_copy(x_buf, out_hbm.at[b, pl.ds(row, CHUNK_ROWS)])


def kernel(x, pos_embed):
    mesh = plsc.VectorSubcoreMesh(
        core_axis_name="c", subcore_axis_name="s",
        num_cores=NUM_CORES, num_subcores=NUM_SUBCORES,
    )
    return pl.kernel(
        _sc_body,
        out_type=jax.ShapeDtypeStruct((BATCH, SEQ_LEN, D_MODEL), jnp.float32),
        mesh=mesh,
        scratch_types=[
            pltpu.VMEM((CHUNK_ROWS, D_MODEL), jnp.float32),
            pltpu.VMEM((CHUNK_ROWS, D_MODEL), jnp.float32),
            pltpu.SemaphoreType.DMA,
        ],
    )(x, pos_embed)


# speedup vs baseline: 1.4764x; 1.4764x over previous
import jax
import jax.numpy as jnp
from jax import lax
from jax.experimental import pallas as pl
from jax.experimental.pallas import tpu as pltpu
from jax.experimental.pallas import tpu_sc as plsc

BATCH = 4
SEQ_LEN = 8192
D_MODEL = 1024

NUM_CORES = 2
NUM_SUBCORES = 16
NUM_WORKERS = NUM_CORES * NUM_SUBCORES
ROWS_PER_WORKER = SEQ_LEN // NUM_WORKERS
CHUNK_ROWS = 32
NUM_CHUNKS = ROWS_PER_WORKER // CHUNK_ROWS
NUM_STEPS = NUM_CHUNKS * BATCH
LANES = 16
GROUPS_PER_ROW = D_MODEL // LANES
GROUPS_PER_CHUNK = CHUNK_ROWS * GROUPS_PER_ROW


def _sc_body(x_hbm, pe_hbm, out_hbm, pe_buf, xb0, xb1, ls0, ls1, ss0, ss1):
    wid = lax.axis_index("s") * NUM_CORES + lax.axis_index("c")
    base = wid * ROWS_PER_WORKER

    xbufs = (xb0, xb1)
    ld_sems = (ls0, ls1)
    st_sems = (ss0, ss1)

    def step_coords(c):
        j, b = divmod(c, BATCH)
        return base + j * CHUNK_ROWS, b

    ld_descs = [None, None]
    st_descs = [None, None]

    row0, b0 = step_coords(0)
    ld_descs[0] = pltpu.async_copy(
        x_hbm.at[b0, pl.ds(row0, CHUNK_ROWS)], xbufs[0], ld_sems[0]
    )

    for c in range(NUM_STEPS):
        i = c % 2
        row, b = step_coords(c)
        if b == 0:
            pltpu.sync_copy(pe_hbm.at[pl.ds(row, CHUNK_ROWS)], pe_buf)
        if c + 1 < NUM_STEPS:
            ni = (c + 1) % 2
            rown, bn = step_coords(c + 1)
            if st_descs[ni] is not None:
                st_descs[ni].wait()
            ld_descs[ni] = pltpu.async_copy(
                x_hbm.at[bn, pl.ds(rown, CHUNK_ROWS)], xbufs[ni], ld_sems[ni]
            )
        ld_descs[i].wait()
        xbuf = xbufs[i]

        @plsc.parallel_loop(0, GROUPS_PER_CHUNK, unroll=8)
        def _(g):
            r = g >> 6
            k = (g & (GROUPS_PER_ROW - 1)) * LANES
            xbuf[r, pl.ds(k, LANES)] = (
                xbuf[r, pl.ds(k, LANES)] + pe_buf[r, pl.ds(k, LANES)]
            )

        st_descs[i] = pltpu.async_copy(
            xbuf, out_hbm.at[b, pl.ds(row, CHUNK_ROWS)], st_sems[i]
        )

    st_descs[0].wait()
    st_descs[1].wait()


def kernel(x, pos_embed):
    mesh = plsc.VectorSubcoreMesh(
        core_axis_name="c", subcore_axis_name="s",
        num_cores=NUM_CORES, num_subcores=NUM_SUBCORES,
    )
    return pl.kernel(
        _sc_body,
        out_type=jax.ShapeDtypeStruct((BATCH, SEQ_LEN, D_MODEL), jnp.float32),
        mesh=mesh,
        scratch_types=[
            pltpu.VMEM((CHUNK_ROWS, D_MODEL), jnp.float32),
            pltpu.VMEM((CHUNK_ROWS, D_MODEL), jnp.float32),
            pltpu.VMEM((CHUNK_ROWS, D_MODEL), jnp.float32),
            pltpu.SemaphoreType.DMA,
            pltpu.SemaphoreType.DMA,
            pltpu.SemaphoreType.DMA,
            pltpu.SemaphoreType.DMA,
        ],
    )(x, pos_embed)


# speedup vs baseline: 1.6832x; 1.1401x over previous
import jax
import jax.numpy as jnp
from jax import lax
from jax.experimental import pallas as pl
from jax.experimental.pallas import tpu as pltpu
from jax.experimental.pallas import tpu_sc as plsc

BATCH = 4
SEQ_LEN = 8192
D_MODEL = 1024

NUM_CORES = 2
NUM_SUBCORES = 16
NUM_WORKERS = NUM_CORES * NUM_SUBCORES
ROWS_PER_WORKER = SEQ_LEN // NUM_WORKERS
CHUNK_ROWS = 8
NUM_STEPS = ROWS_PER_WORKER // CHUNK_ROWS
LANES = 16
GROUPS_PER_ROW = D_MODEL // LANES
GROUPS_PER_CHUNK = CHUNK_ROWS * GROUPS_PER_ROW


def _sc_body(x_hbm, pe_hbm, out_hbm, bufs0, bufs1, ls0, ls1, ss0, ss1):
    wid = lax.axis_index("s") * NUM_CORES + lax.axis_index("c")
    base = wid * ROWS_PER_WORKER

    buf_sets = (bufs0, bufs1)
    ld_sems = (ls0, ls1)
    st_sems = (ss0, ss1)

    def issue_loads(c, i):
        row = base + c * CHUNK_ROWS
        bufs = buf_sets[i]
        descs = [
            pltpu.async_copy(
                pe_hbm.at[pl.ds(row, CHUNK_ROWS)], bufs[0], ld_sems[i]
            )
        ]
        for b in range(BATCH):
            descs.append(
                pltpu.async_copy(
                    x_hbm.at[b, pl.ds(row, CHUNK_ROWS)], bufs[1 + b], ld_sems[i]
                )
            )
        return descs

    ld_descs = [None, None]
    st_descs = [None, None]

    ld_descs[0] = issue_loads(0, 0)

    for c in range(NUM_STEPS):
        i = c % 2
        row = base + c * CHUNK_ROWS
        bufs = buf_sets[i]
        if c + 1 < NUM_STEPS:
            ni = (c + 1) % 2
            if st_descs[ni] is not None:
                for d in st_descs[ni]:
                    d.wait()
            ld_descs[ni] = issue_loads(c + 1, ni)
        for d in ld_descs[i]:
            d.wait()

        pe_buf, xb0, xb1, xb2, xb3 = bufs

        @plsc.parallel_loop(0, GROUPS_PER_CHUNK, unroll=4)
        def _(g):
            r = g >> 6
            k = (g & (GROUPS_PER_ROW - 1)) * LANES
            sl = pl.ds(k, LANES)
            pe = pe_buf[r, sl]
            xb0[r, sl] = xb0[r, sl] + pe
            xb1[r, sl] = xb1[r, sl] + pe
            xb2[r, sl] = xb2[r, sl] + pe
            xb3[r, sl] = xb3[r, sl] + pe

        st_descs[i] = [
            pltpu.async_copy(
                bufs[1 + b], out_hbm.at[b, pl.ds(row, CHUNK_ROWS)], st_sems[i]
            )
            for b in range(BATCH)
        ]

    for descs in st_descs:
        if descs is not None:
            for d in descs:
                d.wait()


def kernel(x, pos_embed):
    mesh = plsc.VectorSubcoreMesh(
        core_axis_name="c", subcore_axis_name="s",
        num_cores=NUM_CORES, num_subcores=NUM_SUBCORES,
    )
    buf_set = [
        pltpu.VMEM((CHUNK_ROWS, D_MODEL), jnp.float32) for _ in range(1 + BATCH)
    ]
    return pl.kernel(
        _sc_body,
        out_type=jax.ShapeDtypeStruct((BATCH, SEQ_LEN, D_MODEL), jnp.float32),
        mesh=mesh,
        scratch_types=[
            buf_set,
            buf_set,
            pltpu.SemaphoreType.DMA,
            pltpu.SemaphoreType.DMA,
            pltpu.SemaphoreType.DMA,
            pltpu.SemaphoreType.DMA,
        ],
    )(x, pos_embed)


# speedup vs baseline: 1.7116x; 1.0169x over previous
import jax
import jax.numpy as jnp
from jax import lax
from jax.experimental import pallas as pl
from jax.experimental.pallas import tpu as pltpu
from jax.experimental.pallas import tpu_sc as plsc

BATCH = 4
SEQ_LEN = 8192
D_MODEL = 1024

NUM_CORES = 2
NUM_SUBCORES = 16
NUM_WORKERS = NUM_CORES * NUM_SUBCORES
ROWS_PER_WORKER = SEQ_LEN // NUM_WORKERS
CHUNK_ROWS = 8
NUM_STEPS = ROWS_PER_WORKER // CHUNK_ROWS
LANES = 16
GROUPS_PER_ROW = D_MODEL // LANES
GROUPS_PER_CHUNK = CHUNK_ROWS * GROUPS_PER_ROW


NBUF = 3


def _sc_body(x_hbm, pe_hbm, out_hbm, bufs0, bufs1, bufs2,
             ls0, ls1, ls2, ss0, ss1, ss2):
    wid = lax.axis_index("s") * NUM_CORES + lax.axis_index("c")
    base = wid * ROWS_PER_WORKER

    buf_sets = (bufs0, bufs1, bufs2)
    ld_sems = (ls0, ls1, ls2)
    st_sems = (ss0, ss1, ss2)

    def issue_loads(c, i):
        row = base + c * CHUNK_ROWS
        bufs = buf_sets[i]
        descs = [
            pltpu.async_copy(
                pe_hbm.at[pl.ds(row, CHUNK_ROWS)], bufs[0], ld_sems[i]
            )
        ]
        for b in range(BATCH):
            descs.append(
                pltpu.async_copy(
                    x_hbm.at[b, pl.ds(row, CHUNK_ROWS)], bufs[1 + b], ld_sems[i]
                )
            )
        return descs

    ld_descs = [None] * NBUF
    st_descs = [None] * NBUF

    ld_descs[0] = issue_loads(0, 0)
    ld_descs[1] = issue_loads(1, 1)

    for c in range(NUM_STEPS):
        i = c % NBUF
        row = base + c * CHUNK_ROWS
        bufs = buf_sets[i]
        if c + 2 < NUM_STEPS:
            ni = (c + 2) % NBUF
            if st_descs[ni] is not None:
                for d in st_descs[ni]:
                    d.wait()
            ld_descs[ni] = issue_loads(c + 2, ni)
        for d in ld_descs[i]:
            d.wait()

        pe_buf, xb0, xb1, xb2, xb3 = bufs

        @plsc.parallel_loop(0, GROUPS_PER_CHUNK, unroll=4)
        def _(g):
            r = g >> 6
            k = (g & (GROUPS_PER_ROW - 1)) * LANES
            sl = pl.ds(k, LANES)
            pe = pe_buf[r, sl]
            xb0[r, sl] = xb0[r, sl] + pe
            xb1[r, sl] = xb1[r, sl] + pe
            xb2[r, sl] = xb2[r, sl] + pe
            xb3[r, sl] = xb3[r, sl] + pe

        st_descs[i] = [
            pltpu.async_copy(
                bufs[1 + b], out_hbm.at[b, pl.ds(row, CHUNK_ROWS)], st_sems[i]
            )
            for b in range(BATCH)
        ]

    for descs in st_descs:
        if descs is not None:
            for d in descs:
                d.wait()


def kernel(x, pos_embed):
    mesh = plsc.VectorSubcoreMesh(
        core_axis_name="c", subcore_axis_name="s",
        num_cores=NUM_CORES, num_subcores=NUM_SUBCORES,
    )
    buf_set = [
        pltpu.VMEM((CHUNK_ROWS, D_MODEL), jnp.float32) for _ in range(1 + BATCH)
    ]
    return pl.kernel(
        _sc_body,
        out_type=jax.ShapeDtypeStruct((BATCH, SEQ_LEN, D_MODEL), jnp.float32),
        mesh=mesh,
        scratch_types=[
            buf_set,
            buf_set,
            buf_set,
            pltpu.SemaphoreType.DMA,
            pltpu.SemaphoreType.DMA,
            pltpu.SemaphoreType.DMA,
            pltpu.SemaphoreType.DMA,
            pltpu.SemaphoreType.DMA,
            pltpu.SemaphoreType.DMA,
        ],
    )(x, pos_embed)
